# baseline (speedup 1.0000x reference)
import functools

import jax
import jax.numpy as jnp
from jax import lax
from jax.experimental import pallas as pl
from jax.experimental.pallas import tpu as pltpu
from jax.experimental.pallas import tpu_sc as plsc

B, N, D = 4, 4096, 2048
BLK = 256
NBLK = N // BLK
_BIG = 2**30


def _tc_score_body(x_ref, wd_ref, val_ref, idx_ref):
    nb = pl.program_id(1)
    x = x_ref[0]
    acc = x[:, 0:128] * wd_ref[0:1, :]
    for k in range(1, D // 128):
        acc = acc + x[:, k * 128:(k + 1) * 128] * wd_ref[k:k + 1, :]
    s = jnp.sum(acc, axis=1, keepdims=True)
    m = jnp.max(s)
    ii = lax.broadcasted_iota(jnp.int32, (BLK, 1), 0)
    am = jnp.min(jnp.where(s == m, ii, _BIG))
    val_ref[0, 0, nb] = m
    idx_ref[0, 0, nb] = am + nb * BLK


def _tc_score(emb, wd2):
    return pl.pallas_call(
        _tc_score_body,
        grid=(B, NBLK),
        compiler_params=pltpu.CompilerParams(
            dimension_semantics=("parallel", "arbitrary")),
        in_specs=[
            pl.BlockSpec((1, BLK, D), lambda b, n: (b, n, 0)),
            pl.BlockSpec((D // 128, 128), lambda b, n: (0, 0)),
        ],
        out_specs=[
            pl.BlockSpec((1, 1, NBLK), lambda b, n: (b, 0, 0),
                         memory_space=pltpu.SMEM),
            pl.BlockSpec((1, 1, NBLK), lambda b, n: (b, 0, 0),
                         memory_space=pltpu.SMEM),
        ],
        out_shape=[
            jax.ShapeDtypeStruct((B, 1, NBLK), jnp.float32),
            jax.ShapeDtypeStruct((B, 1, NBLK), jnp.int32),
        ],
    )(emb, wd2)


@functools.lru_cache(maxsize=1)
def _make_sc_select():
    info = plsc.get_sparse_core_info()
    nc = info.num_cores
    mesh = plsc.VectorSubcoreMesh(core_axis_name="c", subcore_axis_name="s")

    @functools.partial(
        pl.kernel,
        mesh=mesh,
        compiler_params=pltpu.CompilerParams(needs_layout_passes=False),
        out_type=(
            jax.ShapeDtypeStruct((B, D), jnp.float32),
            jax.ShapeDtypeStruct((B, 16), jnp.int32),
        ),
        scratch_types=[
            pltpu.VMEM((16,), jnp.float32),
            pltpu.VMEM((16,), jnp.int32),
            pltpu.VMEM((16,), jnp.int32),
            pltpu.VMEM((16, D), jnp.float32),
            pltpu.SemaphoreType.DMA,
        ],
    )
    def sc_select(vals_hbm, idxs_hbm, emb_hbm, out_emb, out_idx,
                  vals_v, idx_v, gidx_v, rows_v, sem):
        wid = lax.axis_index("s") * nc + lax.axis_index("c")

        @pl.when(wid < B)
        def _():
            b = wid
            pltpu.sync_copy(vals_hbm.at[b], vals_v)
            pltpu.sync_copy(idxs_hbm.at[b], idx_v)
            v = vals_v[...]
            ix = idx_v[...]
            m = jnp.max(v)
            gi = jnp.min(jnp.where(v == m, ix, _BIG))
            idx_v[...] = jnp.full((16,), gi, dtype=jnp.int32)
            gidx_v[...] = jnp.full((16,), gi + b * N, dtype=jnp.int32)
            pltpu.async_copy(emb_hbm.at[gidx_v], rows_v, sem).wait()
            pltpu.sync_copy(rows_v.at[0], out_emb.at[b])
            pltpu.sync_copy(idx_v, out_idx.at[b])

    return sc_select


def kernel(token_embeddings, W, b):
    del b
    wd2 = (W[1] - W[0]).reshape(D // 128, 128)
    vals, idxs = _tc_score(token_embeddings, wd2)
    vals = vals.reshape(B, NBLK)
    idxs = idxs.reshape(B, NBLK)
    emb_flat = token_embeddings.reshape(B * N, D)
    sel_emb, idx16 = _make_sc_select()(vals, idxs, emb_flat)
    return (sel_emb, idx16[:, 0])


# speedup vs baseline: 1.4156x; 1.4156x over previous
import functools

import jax
import jax.numpy as jnp
from jax import lax
from jax.experimental import pallas as pl
from jax.experimental.pallas import tpu as pltpu
from jax.experimental.pallas import tpu_sc as plsc

B, N, D = 4, 4096, 2048
BLK = 1024
NBLK = N // BLK
NCAND = 16
_BIG = 2**30


def _tc_score_body(x_ref, wd_ref, val_ref, idx_ref):
    nb = pl.program_id(1)
    x = x_ref[0]
    acc = x[:, 0:128] * wd_ref[0:1, :]
    for k in range(1, D // 128):
        acc = acc + x[:, k * 128:(k + 1) * 128] * wd_ref[k:k + 1, :]
    s = jnp.sum(acc, axis=1, keepdims=True)
    m = jnp.max(s)
    ii = lax.broadcasted_iota(jnp.int32, (BLK, 1), 0)
    am = jnp.min(jnp.where(s == m, ii, _BIG))

    @pl.when(nb == 0)
    def _():
        for k in range(NBLK, NCAND):
            val_ref[0, 0, k] = float("-inf")
            idx_ref[0, 0, k] = _BIG

    val_ref[0, 0, nb] = m
    idx_ref[0, 0, nb] = am + nb * BLK


def _tc_score(emb, wd2):
    return pl.pallas_call(
        _tc_score_body,
        grid=(B, NBLK),
        compiler_params=pltpu.CompilerParams(
            dimension_semantics=("parallel", "arbitrary")),
        in_specs=[
            pl.BlockSpec((1, BLK, D), lambda b, n: (b, n, 0)),
            pl.BlockSpec((D // 128, 128), lambda b, n: (0, 0)),
        ],
        out_specs=[
            pl.BlockSpec((1, 1, NCAND), lambda b, n: (b, 0, 0),
                         memory_space=pltpu.SMEM),
            pl.BlockSpec((1, 1, NCAND), lambda b, n: (b, 0, 0),
                         memory_space=pltpu.SMEM),
        ],
        out_shape=[
            jax.ShapeDtypeStruct((B, 1, NCAND), jnp.float32),
            jax.ShapeDtypeStruct((B, 1, NCAND), jnp.int32),
        ],
    )(emb, wd2)


@functools.lru_cache(maxsize=1)
def _make_sc_select():
    info = plsc.get_sparse_core_info()
    nc = info.num_cores
    mesh = plsc.VectorSubcoreMesh(core_axis_name="c", subcore_axis_name="s")

    @functools.partial(
        pl.kernel,
        mesh=mesh,
        compiler_params=pltpu.CompilerParams(needs_layout_passes=False),
        out_type=(
            jax.ShapeDtypeStruct((B, D), jnp.float32),
            jax.ShapeDtypeStruct((B, 16), jnp.int32),
        ),
        scratch_types=[
            pltpu.VMEM((16,), jnp.float32),
            pltpu.VMEM((16,), jnp.int32),
            pltpu.VMEM((16,), jnp.int32),
            pltpu.VMEM((16, D), jnp.float32),
            pltpu.SemaphoreType.DMA,
        ],
    )
    def sc_select(vals_hbm, idxs_hbm, emb_hbm, out_emb, out_idx,
                  vals_v, idx_v, gidx_v, rows_v, sem):
        wid = lax.axis_index("s") * nc + lax.axis_index("c")

        @pl.when(wid < B)
        def _():
            b = wid
            pltpu.sync_copy(vals_hbm.at[b], vals_v)
            pltpu.sync_copy(idxs_hbm.at[b], idx_v)
            v = vals_v[...]
            ix = idx_v[...]
            m = jnp.max(v)
            gi = jnp.min(jnp.where(v == m, ix, _BIG))
            idx_v[...] = jnp.full((16,), gi, dtype=jnp.int32)
            gidx_v[...] = jnp.full((16,), gi + b * N, dtype=jnp.int32)
            pltpu.async_copy(emb_hbm.at[gidx_v], rows_v, sem).wait()
            pltpu.sync_copy(rows_v.at[0], out_emb.at[b])
            pltpu.sync_copy(idx_v, out_idx.at[b])

    return sc_select


def kernel(token_embeddings, W, b):
    del b
    wd2 = (W[1] - W[0]).reshape(D // 128, 128)
    vals, idxs = _tc_score(token_embeddings, wd2)
    vals = vals.reshape(B, NCAND)
    idxs = idxs.reshape(B, NCAND)
    emb_flat = token_embeddings.reshape(B * N, D)
    sel_emb, idx16 = _make_sc_select()(vals, idxs, emb_flat)
    return (sel_emb, idx16[:, 0])
